# speedup vs baseline: 1.2832x; 1.2832x over previous
import functools

import jax
import jax.numpy as jnp
from jax import lax
from jax.experimental import pallas as pl
from jax.experimental.pallas import tpu as pltpu
from jax.experimental.pallas import tpu_sc as plsc

_BATCH = 4096
_HIST = 50
_EMB = 256
_ROWS = _BATCH * _HIST
_NW = 32
_ROWS_PER_W = _ROWS // _NW
_CHUNK = 128
_NCHUNK = _ROWS_PER_W // _CHUNK


def _gather_body(idx_hbm, table_hbm, out_hbm, idx_v, buf0, buf1, sem0, sem1):
    wid = lax.axis_index("s") * 2 + lax.axis_index("c")
    row_base = wid * _ROWS_PER_W

    pltpu.sync_copy(idx_hbm.at[wid], idx_v)

    def start_gather(c, buf, sem):
        pltpu.async_copy(table_hbm.at[idx_v.at[c]], buf, sem)

    def finish(c, buf, sem):
        pltpu.make_async_copy(table_hbm.at[idx_v.at[c]], buf, sem).wait()
        pltpu.sync_copy(buf, out_hbm.at[pl.ds(row_base + c * _CHUNK, _CHUNK)])

    start_gather(0, buf0, sem0)
    start_gather(1, buf1, sem1)

    def body(g, carry):
        c0 = 2 * g
        c1 = c0 + 1
        finish(c0, buf0, sem0)

        @pl.when(c0 + 2 < _NCHUNK)
        def _():
            start_gather(c0 + 2, buf0, sem0)

        finish(c1, buf1, sem1)

        @pl.when(c1 + 2 < _NCHUNK)
        def _():
            start_gather(c1 + 2, buf1, sem1)

        return carry

    lax.fori_loop(0, _NCHUNK // 2, body, 0)


@jax.jit
def _lookup(indices_3d, ent_embeds):
    mesh = plsc.VectorSubcoreMesh(core_axis_name="c", subcore_axis_name="s")
    run = functools.partial(
        pl.kernel,
        out_type=jax.ShapeDtypeStruct((_ROWS, _EMB), jnp.float32),
        mesh=mesh,
        scratch_types=[
            pltpu.VMEM((_NCHUNK, _CHUNK), jnp.int32),
            pltpu.VMEM((_CHUNK, _EMB), jnp.float32),
            pltpu.VMEM((_CHUNK, _EMB), jnp.float32),
            pltpu.SemaphoreType.DMA,
            pltpu.SemaphoreType.DMA,
        ],
    )(_gather_body)
    return run(indices_3d, ent_embeds)


def kernel(indices, ent_embeds):
    idx3 = indices.reshape(_NW, _NCHUNK, _CHUNK).astype(jnp.int32)
    out = _lookup(idx3, ent_embeds)
    return out.reshape(_BATCH, _HIST, _EMB)
